# baseline (speedup 1.0000x reference)
import functools

import jax
import jax.numpy as jnp
from jax import lax
from jax.experimental import pallas as pl
from jax.experimental.pallas import tpu as pltpu
from jax.experimental.pallas import tpu_sc as plsc

_INV_EXT = 1.0 / 0.06
_K = 10
_NNEI = 32
_CHUNK = 64
_NBUF = 8
_WPAD = 384


def _fast_sqrt(d2):
    d2 = jnp.maximum(d2, 1e-30)
    i = plsc.bitcast(d2, jnp.int32)
    i = jnp.int32(0x5F3759DF) - (i >> 1)
    y = plsc.bitcast(i, jnp.float32)
    for _ in range(3):
        y = y * (1.5 - 0.5 * d2 * y * y)
    return d2 * y


def _sc_gather_and_weights(feat_tab, idx_flat, sxa, sya, sza, qxa, qya, qza,
                           kp48, rows_pad, n_queries_pad, num_cores,
                           num_workers):
    rows_per_worker = rows_pad // num_workers
    q_per_worker = rows_per_worker // _NNEI
    n_sec = rows_per_worker // _CHUNK
    n_outer = n_sec // _NBUF
    npts = sxa.shape[0]
    mesh = plsc.VectorSubcoreMesh(core_axis_name="c", subcore_axis_name="s")

    @functools.partial(
        pl.kernel,
        mesh=mesh,
        compiler_params=pltpu.CompilerParams(needs_layout_passes=False),
        out_type=(
            jax.ShapeDtypeStruct((rows_pad, 128), jnp.float32),
            jax.ShapeDtypeStruct((n_queries_pad, _WPAD), jnp.float32),
        ),
        scratch_types=(
            [pltpu.VMEM((rows_per_worker,), jnp.int32),
             pltpu.VMEM((npts,), jnp.float32),
             pltpu.VMEM((npts,), jnp.float32),
             pltpu.VMEM((npts,), jnp.float32),
             pltpu.VMEM((q_per_worker,), jnp.float32),
             pltpu.VMEM((q_per_worker,), jnp.float32),
             pltpu.VMEM((q_per_worker,), jnp.float32),
             pltpu.VMEM((48,), jnp.float32),
             pltpu.VMEM((8, _WPAD), jnp.float32)]
            + [pltpu.VMEM((_CHUNK, 128), jnp.float32) for _ in range(_NBUF)]
            + [pltpu.SemaphoreType.DMA for _ in range(2 * _NBUF)]
        ),
    )
    def sc_kernel(feat_hbm, idx_hbm, sx_h, sy_h, sz_h, qx_h, qy_h, qz_h,
                  kp_h, gath_hbm, w_hbm, idx_v, sx_v, sy_v, sz_v,
                  qx_v, qy_v, qz_v, kp_v, w_buf, *bufs_and_sems):
        fbs = bufs_and_sems[:_NBUF]
        sgs = bufs_and_sems[_NBUF:2 * _NBUF]
        sos = bufs_and_sems[2 * _NBUF:]
        wid = lax.axis_index("s") * num_cores + lax.axis_index("c")
        row_base = wid * rows_per_worker
        q_base = wid * q_per_worker
        pltpu.sync_copy(idx_hbm.at[pl.ds(row_base, rows_per_worker)], idx_v)
        pltpu.sync_copy(sx_h, sx_v)
        pltpu.sync_copy(sy_h, sy_v)
        pltpu.sync_copy(sz_h, sz_v)
        pltpu.sync_copy(qx_h.at[pl.ds(q_base, q_per_worker)], qx_v)
        pltpu.sync_copy(qy_h.at[pl.ds(q_base, q_per_worker)], qy_v)
        pltpu.sync_copy(qz_h.at[pl.ds(q_base, q_per_worker)], qz_v)
        pltpu.sync_copy(kp_h, kp_v)
        lanes = lax.iota(jnp.int32, 16)

        kpx = []
        kpy = []
        kpz = []
        for k in range(_K):
            kk = jnp.full((16,), k + 1, jnp.int32)
            kpx.append(plsc.load_gather(kp_v, [kk]))
            kpy.append(plsc.load_gather(kp_v, [kk + 16]))
            kpz.append(plsc.load_gather(kp_v, [kk + 32]))

        def fire_gather(sec, ring):
            pltpu.async_copy(
                feat_hbm.at[idx_v.at[pl.ds(sec * _CHUNK, _CHUNK)]],
                fbs[ring], sgs[ring])

        def drain_out(ring):
            pltpu.make_async_copy(
                fbs[ring], gath_hbm.at[pl.ds(row_base, _CHUNK)],
                sos[ring]).wait()

        def wait_gather(ring):
            pltpu.make_async_copy(
                feat_hbm.at[pl.ds(0, _CHUNK)], fbs[ring], sgs[ring]).wait()

        def compute_weights(sec, quarter):
            def grp_body(g, carry):
                goff = sec * _CHUNK + g * 16
                nbr = idx_v[pl.ds(goff, 16)]
                qidx = ((row_base + goff + lanes) >> 5) - q_base
                px = plsc.load_gather(sx_v, [nbr]) - plsc.load_gather(qx_v, [qidx])
                py = plsc.load_gather(sy_v, [nbr]) - plsc.load_gather(qy_v, [qidx])
                pz = plsc.load_gather(sz_v, [nbr]) - plsc.load_gather(qz_v, [qidx])
                qloc = quarter * 2 + (g >> 1)
                jloc = (g & 1) * 16
                for k in range(_K):
                    dx = px - kpx[k]
                    dy = py - kpy[k]
                    dz = pz - kpz[k]
                    d2 = dx * dx + dy * dy + dz * dz
                    w = jnp.maximum(1.0 - _fast_sqrt(d2) * _INV_EXT, 0.0)
                    w_buf[qloc, pl.ds(k * 32 + jloc, 16)] = w
                return carry

            lax.fori_loop(0, _CHUNK // 16, grp_body, 0)

        half = _NBUF // 2
        for s0 in range(half):
            fire_gather(s0, s0)

        def outer(t, carry):
            for b in range(_NBUF):
                sec = t * _NBUF + b
                nxt = (b + half) % _NBUF
                if b < half:
                    @pl.when(t > 0)
                    def _():
                        drain_out(nxt)
                    fire_gather(sec + half, nxt)
                else:
                    @pl.when(t < n_outer - 1)
                    def _():
                        drain_out(nxt)
                        fire_gather(sec + half, nxt)

                    @pl.when(t == n_outer - 1)
                    def _():
                        drain_out(nxt)
                wait_gather(b)
                compute_weights(sec, b % 4)
                pltpu.async_copy(
                    fbs[b],
                    gath_hbm.at[pl.ds(row_base + sec * _CHUNK, _CHUNK)],
                    sos[b])
                if b % 4 == 3:
                    pltpu.sync_copy(
                        w_buf,
                        w_hbm.at[pl.ds(q_base + (sec - 3) * (_CHUNK // _NNEI), 8)])
            return carry

        lax.fori_loop(0, n_outer, outer, 0)
        for b in range(half, _NBUF):
            drain_out(b)

    return sc_kernel(feat_tab, idx_flat, sxa, sya, sza, qxa, qya, qza, kp48)


def _tc_compute(gathered, wmat, weights, n, block_q):
    cin = weights.shape[1]
    cout = weights.shape[2]

    def body(g_ref, w_ref, wt_ref, o_ref):
        feats = g_ref[...].reshape(block_q, _NNEI, cin)
        acc = jnp.zeros((block_q, cout), jnp.float32)
        for k in range(_K):
            wk = w_ref[:, k * 32:k * 32 + 32]
            weighted = jnp.sum(wk[:, :, None] * feats, axis=1)
            acc = acc + jnp.dot(weighted, wt_ref[k],
                                preferred_element_type=jnp.float32)
        o_ref[...] = acc

    return pl.pallas_call(
        body,
        grid=(n // block_q,),
        in_specs=[
            pl.BlockSpec((block_q * _NNEI, cin), lambda i: (i, 0)),
            pl.BlockSpec((block_q, _WPAD), lambda i: (i, 0)),
            pl.BlockSpec((_K, cin, cout), lambda i: (0, 0, 0)),
        ],
        out_specs=pl.BlockSpec((block_q, cout), lambda i: (i, 0)),
        out_shape=jax.ShapeDtypeStruct((n, cout), jnp.float32),
        compiler_params=pltpu.CompilerParams(
            dimension_semantics=("arbitrary",),
        ),
    )(gathered, wmat, weights)


def kernel(q_pts, s_pts, x, kernel_points, weights, neighb_inds):
    n, cin = x.shape
    n_nei = neighb_inds.shape[1]

    info = plsc.get_sparse_core_info()
    num_workers = info.num_cores * info.num_subcores
    rows = n * n_nei
    sec_rows = _NBUF * _CHUNK
    per_worker = -(-rows // (num_workers * sec_rows)) * sec_rows
    rows_pad = num_workers * per_worker
    n_queries_pad = rows_pad // n_nei

    feat_tab = jnp.concatenate([x, jnp.zeros((1, cin), jnp.float32)], axis=0)

    flat_idx = jnp.concatenate(
        [neighb_inds.reshape(-1),
         jnp.full((rows_pad - rows,), n, jnp.int32)])

    npts = -(-(n + 1) // 128) * 128

    def col(a, i, length, shadow):
        parts = [a[:, i]]
        if shadow is not None:
            parts.append(jnp.full((1,), shadow, jnp.float32))
        parts.append(jnp.zeros((length - sum(p.shape[0] for p in parts),),
                               jnp.float32))
        return jnp.concatenate(parts)

    sxa = col(s_pts, 0, npts, 1e6)
    sya = col(s_pts, 1, npts, 1e6)
    sza = col(s_pts, 2, npts, 1e6)
    qxa = col(q_pts, 0, n_queries_pad, None)
    qya = col(q_pts, 1, n_queries_pad, None)
    qza = col(q_pts, 2, n_queries_pad, None)
    kp48 = jnp.concatenate([
        jnp.pad(kernel_points[:, 0], (1, 16 - _K - 1)),
        jnp.pad(kernel_points[:, 1], (1, 16 - _K - 1)),
        jnp.pad(kernel_points[:, 2], (1, 16 - _K - 1)),
    ])

    gathered, wmat = _sc_gather_and_weights(
        feat_tab, flat_idx, sxa, sya, sza, qxa, qya, qza, kp48,
        rows_pad, n_queries_pad, info.num_cores, num_workers)

    return _tc_compute(gathered, wmat, weights, n, block_q=200)


# speedup vs baseline: 1.5659x; 1.5659x over previous
import functools

import jax
import jax.numpy as jnp
from jax import lax
from jax.experimental import pallas as pl
from jax.experimental.pallas import tpu as pltpu
from jax.experimental.pallas import tpu_sc as plsc

_INV_EXT = 1.0 / 0.06
_K = 10
_NNEI = 32
_CHUNK = 64
_NBUF = 8
_WPAD = 384


def _fast_sqrt(d2):
    d2 = jnp.maximum(d2, 1e-30)
    i = plsc.bitcast(d2, jnp.int32)
    i = jnp.int32(0x5F3759DF) - (i >> 1)
    y = plsc.bitcast(i, jnp.float32)
    for _ in range(3):
        y = y * (1.5 - 0.5 * d2 * y * y)
    return d2 * y


def _sc_gather_and_weights(feat_tab, idx_flat, sxa, sya, sza, qxa, qya, qza,
                           kp48, rows_pad, n_queries_pad, num_cores,
                           num_workers):
    rows_per_worker = rows_pad // num_workers
    q_per_worker = rows_per_worker // _NNEI
    n_sec = rows_per_worker // _CHUNK
    n_outer = n_sec // _NBUF
    npts = sxa.shape[0]
    mesh = plsc.VectorSubcoreMesh(core_axis_name="c", subcore_axis_name="s")

    @functools.partial(
        pl.kernel,
        mesh=mesh,
        compiler_params=pltpu.CompilerParams(needs_layout_passes=False),
        out_type=(
            jax.ShapeDtypeStruct((rows_pad, 128), jnp.float32),
            jax.ShapeDtypeStruct((n_queries_pad, _WPAD), jnp.float32),
        ),
        scratch_types=(
            [pltpu.VMEM((rows_per_worker,), jnp.int32),
             pltpu.VMEM((npts,), jnp.float32),
             pltpu.VMEM((npts,), jnp.float32),
             pltpu.VMEM((npts,), jnp.float32),
             pltpu.VMEM((q_per_worker,), jnp.float32),
             pltpu.VMEM((q_per_worker,), jnp.float32),
             pltpu.VMEM((q_per_worker,), jnp.float32),
             pltpu.VMEM((48,), jnp.float32),
             pltpu.VMEM((8, _WPAD), jnp.float32)]
            + [pltpu.VMEM((_CHUNK, 128), jnp.float32) for _ in range(_NBUF)]
            + [pltpu.SemaphoreType.DMA for _ in range(2 * _NBUF)]
        ),
    )
    def sc_kernel(feat_hbm, idx_hbm, sx_h, sy_h, sz_h, qx_h, qy_h, qz_h,
                  kp_h, gath_hbm, w_hbm, idx_v, sx_v, sy_v, sz_v,
                  qx_v, qy_v, qz_v, kp_v, w_buf, *bufs_and_sems):
        fbs = bufs_and_sems[:_NBUF]
        sgs = bufs_and_sems[_NBUF:2 * _NBUF]
        sos = bufs_and_sems[2 * _NBUF:]
        wid = lax.axis_index("s") * num_cores + lax.axis_index("c")
        row_base = wid * rows_per_worker
        q_base = wid * q_per_worker
        pltpu.sync_copy(idx_hbm.at[pl.ds(row_base, rows_per_worker)], idx_v)
        pltpu.sync_copy(sx_h, sx_v)
        pltpu.sync_copy(sy_h, sy_v)
        pltpu.sync_copy(sz_h, sz_v)
        pltpu.sync_copy(qx_h.at[pl.ds(q_base, q_per_worker)], qx_v)
        pltpu.sync_copy(qy_h.at[pl.ds(q_base, q_per_worker)], qy_v)
        pltpu.sync_copy(qz_h.at[pl.ds(q_base, q_per_worker)], qz_v)
        pltpu.sync_copy(kp_h, kp_v)
        lanes = lax.iota(jnp.int32, 16)

        kpx = []
        kpy = []
        kpz = []
        for k in range(_K):
            kk = jnp.full((16,), k + 1, jnp.int32)
            kpx.append(plsc.load_gather(kp_v, [kk]))
            kpy.append(plsc.load_gather(kp_v, [kk + 16]))
            kpz.append(plsc.load_gather(kp_v, [kk + 32]))

        def fire_gather(sec, ring):
            pltpu.async_copy(
                feat_hbm.at[idx_v.at[pl.ds(sec * _CHUNK, _CHUNK)]],
                fbs[ring], sgs[ring])

        def drain_out(ring):
            pltpu.make_async_copy(
                fbs[ring], gath_hbm.at[pl.ds(row_base, _CHUNK)],
                sos[ring]).wait()

        def wait_gather(ring):
            pltpu.make_async_copy(
                feat_hbm.at[pl.ds(0, _CHUNK)], fbs[ring], sgs[ring]).wait()

        def compute_weights(sec, quarter):
            def grp_body(g, carry):
                goff = sec * _CHUNK + g * 16
                nbr = idx_v[pl.ds(goff, 16)]
                qidx = ((row_base + goff + lanes) >> 5) - q_base
                px = plsc.load_gather(sx_v, [nbr]) - plsc.load_gather(qx_v, [qidx])
                py = plsc.load_gather(sy_v, [nbr]) - plsc.load_gather(qy_v, [qidx])
                pz = plsc.load_gather(sz_v, [nbr]) - plsc.load_gather(qz_v, [qidx])
                qloc = quarter * 2 + (g >> 1)
                jloc = (g & 1) * 16
                for k in range(_K):
                    dx = px - kpx[k]
                    dy = py - kpy[k]
                    dz = pz - kpz[k]
                    d2 = dx * dx + dy * dy + dz * dz
                    w = jnp.maximum(1.0 - _fast_sqrt(d2) * _INV_EXT, 0.0)
                    w_buf[qloc, pl.ds(k * 32 + jloc, 16)] = w
                return carry

            lax.fori_loop(0, _CHUNK // 16, grp_body, 0)

        half = _NBUF // 2
        for s0 in range(half):
            fire_gather(s0, s0)

        def outer(t, carry):
            for b in range(_NBUF):
                sec = t * _NBUF + b
                nxt = (b + half) % _NBUF
                if b < half:
                    @pl.when(t > 0)
                    def _():
                        drain_out(nxt)
                    fire_gather(sec + half, nxt)
                else:
                    @pl.when(t < n_outer - 1)
                    def _():
                        drain_out(nxt)
                        fire_gather(sec + half, nxt)

                    @pl.when(t == n_outer - 1)
                    def _():
                        drain_out(nxt)
                wait_gather(b)
                compute_weights(sec, b % 4)
                pltpu.async_copy(
                    fbs[b],
                    gath_hbm.at[pl.ds(row_base + sec * _CHUNK, _CHUNK)],
                    sos[b])
                if b % 4 == 3:
                    pltpu.sync_copy(
                        w_buf,
                        w_hbm.at[pl.ds(q_base + (sec - 3) * (_CHUNK // _NNEI), 8)])
            return carry

        lax.fori_loop(0, n_outer, outer, 0)
        for b in range(half, _NBUF):
            drain_out(b)

    return sc_kernel(feat_tab, idx_flat, sxa, sya, sza, qxa, qya, qza, kp48)


def _tc_compute(gathered, wmat, weights, n, block_q):
    cin = weights.shape[1]
    cout = weights.shape[2]

    def body(g_ref, w_ref, wt_ref, o_ref):
        feats = g_ref[...].reshape(block_q, _NNEI, cin)
        w3 = w_ref[:, :_K * 32].reshape(block_q, _K, _NNEI)
        weighted = jax.lax.dot_general(
            w3, feats, (((2,), (1,)), ((0,), (0,))),
            preferred_element_type=jnp.float32)
        acc = jnp.zeros((block_q, cout), jnp.float32)
        for k in range(_K):
            acc = acc + jnp.dot(weighted[:, k, :], wt_ref[k],
                                preferred_element_type=jnp.float32)
        o_ref[...] = acc

    return pl.pallas_call(
        body,
        grid=(n // block_q,),
        in_specs=[
            pl.BlockSpec((block_q * _NNEI, cin), lambda i: (i, 0)),
            pl.BlockSpec((block_q, _WPAD), lambda i: (i, 0)),
            pl.BlockSpec((_K, cin, cout), lambda i: (0, 0, 0)),
        ],
        out_specs=pl.BlockSpec((block_q, cout), lambda i: (i, 0)),
        out_shape=jax.ShapeDtypeStruct((n, cout), jnp.float32),
        compiler_params=pltpu.CompilerParams(
            dimension_semantics=("arbitrary",),
        ),
    )(gathered, wmat, weights)


def kernel(q_pts, s_pts, x, kernel_points, weights, neighb_inds):
    n, cin = x.shape
    n_nei = neighb_inds.shape[1]

    info = plsc.get_sparse_core_info()
    num_workers = info.num_cores * info.num_subcores
    rows = n * n_nei
    sec_rows = _NBUF * _CHUNK
    per_worker = -(-rows // (num_workers * sec_rows)) * sec_rows
    rows_pad = num_workers * per_worker
    n_queries_pad = rows_pad // n_nei

    feat_tab = jnp.concatenate([x, jnp.zeros((1, cin), jnp.float32)], axis=0)

    flat_idx = jnp.concatenate(
        [neighb_inds.reshape(-1),
         jnp.full((rows_pad - rows,), n, jnp.int32)])

    npts = -(-(n + 1) // 128) * 128

    def col(a, i, length, shadow):
        parts = [a[:, i]]
        if shadow is not None:
            parts.append(jnp.full((1,), shadow, jnp.float32))
        parts.append(jnp.zeros((length - sum(p.shape[0] for p in parts),),
                               jnp.float32))
        return jnp.concatenate(parts)

    sxa = col(s_pts, 0, npts, 1e6)
    sya = col(s_pts, 1, npts, 1e6)
    sza = col(s_pts, 2, npts, 1e6)
    qxa = col(q_pts, 0, n_queries_pad, None)
    qya = col(q_pts, 1, n_queries_pad, None)
    qza = col(q_pts, 2, n_queries_pad, None)
    kp48 = jnp.concatenate([
        jnp.pad(kernel_points[:, 0], (1, 16 - _K - 1)),
        jnp.pad(kernel_points[:, 1], (1, 16 - _K - 1)),
        jnp.pad(kernel_points[:, 2], (1, 16 - _K - 1)),
    ])

    gathered, wmat = _sc_gather_and_weights(
        feat_tab, flat_idx, sxa, sya, sza, qxa, qya, qza, kp48,
        rows_pad, n_queries_pad, info.num_cores, num_workers)

    return _tc_compute(gathered, wmat, weights, n, block_q=200)
